# Initial kernel scaffold; baseline (speedup 1.0000x reference)
import jax
import jax.numpy as jnp
from jax.experimental import pallas as pl


def kernel(sequences, expert_weights, expert_biases, gating_w, gating_b):
    raise NotImplementedError("write your pallas kernel here")


# speedup vs baseline: 2.3363x; 2.3363x over previous
import functools

import jax
import jax.numpy as jnp
from jax.experimental import pallas as pl


N_EXP = 8
TOP_K = 2


def _gating_body(x_ref, gw_ref, gb_ref, c_ref):
    x = x_ref[...]
    logits = jax.lax.dot_general(
        x, gw_ref[...], (((1,), (1,)), ((), ())),
        precision=jax.lax.Precision.DEFAULT,
        preferred_element_type=jnp.float32,
    ) + gb_ref[...]
    m = jnp.max(logits, axis=-1, keepdims=True)
    p = jnp.exp(logits - m)
    s = p / jnp.sum(p, axis=-1, keepdims=True)
    ii = jax.lax.broadcasted_iota(jnp.int32, s.shape, 1)
    m1 = jnp.max(s, axis=-1, keepdims=True)
    i1 = jnp.min(jnp.where(s == m1, ii, N_EXP), axis=-1, keepdims=True)
    s2 = jnp.where(ii == i1, -jnp.inf, s)
    m2 = jnp.max(s2, axis=-1, keepdims=True)
    i2 = jnp.min(jnp.where(s2 == m2, ii, N_EXP), axis=-1, keepdims=True)
    sel = (ii == i1) | (ii == i2)
    c_ref[...] = jnp.where(sel, s, 0.0) * (1.0 / TOP_K)


def _moe_body(c_ref, x_ref, w_ref, b_ref, out_ref):
    e = pl.program_id(1)
    c = c_ref[...]
    ee = jax.lax.broadcasted_iota(jnp.int32, c.shape, 1)
    c_col = jnp.sum(jnp.where(ee == e, c, 0.0), axis=1, keepdims=True)
    t = jnp.dot(x_ref[...], w_ref[0], preferred_element_type=jnp.float32)
    t = (t + b_ref[0]) * c_col

    @pl.when(e == 0)
    def _init():
        out_ref[...] = t

    @pl.when(e != 0)
    def _acc():
        out_ref[...] += t


def kernel(sequences, expert_weights, expert_biases, gating_w, gating_b):
    n, p, d = sequences.shape
    tokens = n * p
    d_inner = expert_biases.shape[-1]
    x = sequences.reshape(tokens, d)

    c = pl.pallas_call(
        _gating_body,
        out_shape=jax.ShapeDtypeStruct((tokens, N_EXP), jnp.float32),
    )(x, gating_w, gating_b.reshape(1, N_EXP))

    tn = 512
    n_tiles = d_inner // tn
    out = pl.pallas_call(
        _moe_body,
        grid=(n_tiles, N_EXP),
        in_specs=[
            pl.BlockSpec((tokens, N_EXP), lambda ni, e: (0, 0)),
            pl.BlockSpec((tokens, d), lambda ni, e: (0, 0)),
            pl.BlockSpec((1, d, tn), lambda ni, e: (e, 0, ni)),
            pl.BlockSpec((1, 1, tn), lambda ni, e: (e, 0, ni)),
        ],
        out_specs=pl.BlockSpec((tokens, tn), lambda ni, e: (0, ni)),
        out_shape=jax.ShapeDtypeStruct((tokens, d_inner), jnp.float32),
    )(c, x, expert_weights, expert_biases.reshape(N_EXP, 1, d_inner))

    return out.reshape(n, p, d_inner)
